# Initial kernel scaffold; baseline (speedup 1.0000x reference)
import jax
import jax.numpy as jnp
from jax.experimental import pallas as pl


def kernel(center, context, negatives, input_table, output_table):
    raise NotImplementedError("write your pallas kernel here")


# speedup vs baseline: 4.0036x; 4.0036x over previous
import functools

import jax
import jax.numpy as jnp
from jax import lax
from jax.experimental import pallas as pl
from jax.experimental.pallas import tpu as pltpu
from jax.experimental.pallas import tpu_sc as plsc

VOCAB = 1000000
DIM = 64
B = 16384
K = 20

NC = 2
NS = 16
NW = NC * NS
EW = B // NW
CB = 32
NCH = EW // CB
GPC = CB // 16
NEG_CH = CB * K
NEG_STREAMS = NEG_CH // 128


def _sc_scores_kernel(center, context, neg, in_tab, out_tab,
                      pos_out, neg_out,
                      cen_idx, ctx_idx, neg_idx,
                      cen_rows, pos_rows, neg_rows,
                      pos_sc, neg_sc, sem0, sem1):
    wid = lax.axis_index("s") * NC + lax.axis_index("c")
    base = wid * EW

    pltpu.sync_copy(center.at[pl.ds(base, EW)], cen_idx)
    pltpu.sync_copy(context.at[pl.ds(base, EW)], ctx_idx)
    pltpu.sync_copy(neg.at[pl.ds(base * K, EW * K)], neg_idx)

    sems = (sem0, sem1)

    def fire(j, par):
        descs = [
            pltpu.make_async_copy(
                in_tab.at[cen_idx.at[pl.ds(j * CB, CB)]],
                cen_rows.at[par], sems[par]),
            pltpu.make_async_copy(
                out_tab.at[ctx_idx.at[pl.ds(j * CB, CB)]],
                pos_rows.at[par], sems[par]),
        ]
        for s in range(NEG_STREAMS):
            descs.append(pltpu.make_async_copy(
                out_tab.at[neg_idx.at[pl.ds(j * NEG_CH + s * 128, 128)]],
                neg_rows.at[par, pl.ds(s * 128, 128)], sems[par]))
        for d in descs:
            d.start()
        return descs

    iota = lax.iota(jnp.int32, 16)

    def compute(j, par):
        crows = cen_rows.at[par]
        prows = pos_rows.at[par]
        nrows = neg_rows.at[par]
        for g in range(GPC):
            row_ids = g * 16 + iota
            nbase = row_ids * K

            def body(d, carry):
                dcol = jnp.full((16,), d, dtype=jnp.int32)
                cd = plsc.load_gather(crows, [row_ids, dcol])
                pd = plsc.load_gather(prows, [row_ids, dcol])
                accs = [carry[0] + cd * pd]
                for k in range(K):
                    nd = plsc.load_gather(nrows, [nbase + k, dcol])
                    accs.append(carry[k + 1] + cd * nd)
                return tuple(accs)

            zeros = tuple(jnp.zeros((16,), jnp.float32) for _ in range(K + 1))
            accs = lax.fori_loop(0, DIM, body, zeros)

            off = j * CB + g * 16
            pos_sc[pl.ds(off, 16)] = accs[0]
            for k in range(K):
                neg_sc[k, pl.ds(off, 16)] = accs[k + 1]

    descs = fire(0, 0)
    for j in range(NCH):
        nxt = fire(j + 1, (j + 1) % 2) if j + 1 < NCH else []
        for d in descs:
            d.wait()
        compute(j, j % 2)
        descs = nxt

    pltpu.sync_copy(pos_sc, pos_out.at[pl.ds(base, EW)])
    pltpu.sync_copy(neg_sc, neg_out.at[wid])


@functools.partial(jax.jit, static_argnames=())
def _sc_scores(center, context, neg_flat, in_tab, out_tab):
    mesh = plsc.VectorSubcoreMesh(core_axis_name="c", subcore_axis_name="s")
    return pl.kernel(
        _sc_scores_kernel,
        out_type=(
            jax.ShapeDtypeStruct((B,), jnp.float32),
            jax.ShapeDtypeStruct((NW, K, EW), jnp.float32),
        ),
        mesh=mesh,
        scratch_types=(
            pltpu.VMEM((EW,), jnp.int32),
            pltpu.VMEM((EW,), jnp.int32),
            pltpu.VMEM((EW * K,), jnp.int32),
            pltpu.VMEM((2, CB, DIM), jnp.float32),
            pltpu.VMEM((2, CB, DIM), jnp.float32),
            pltpu.VMEM((2, NEG_CH, DIM), jnp.float32),
            pltpu.VMEM((EW,), jnp.float32),
            pltpu.VMEM((K, EW), jnp.float32),
            pltpu.SemaphoreType.DMA,
            pltpu.SemaphoreType.DMA,
        ),
        compiler_params=pltpu.CompilerParams(needs_layout_passes=False,
                                             use_tc_tiling_on_sc=False),
        name="w2v_sc_scores",
    )(center, context, neg_flat, in_tab, out_tab)


def _tc_loss_kernel(pos_ref, neg_ref, out_ref):
    def ls(x):
        return jnp.minimum(x, 0.0) - jnp.log1p(jnp.exp(-jnp.abs(x)))

    total = jnp.sum(ls(pos_ref[...])) + jnp.sum(ls(-neg_ref[...]))
    out_ref[...] = jnp.full((1, 1), -1.0 / B) * total


def _tc_loss(pos, negs):
    pos2 = pos.reshape(128, 128)
    neg2 = negs.reshape(NW * K, EW)
    out = pl.pallas_call(
        _tc_loss_kernel,
        out_shape=jax.ShapeDtypeStruct((1, 1), jnp.float32),
    )(pos2, neg2)
    return out.reshape(())


def kernel(center, context, negatives, input_table, output_table):
    neg_flat = negatives.reshape(B * K)
    pos_sc, neg_sc = _sc_scores(center.astype(jnp.int32),
                                context.astype(jnp.int32),
                                neg_flat.astype(jnp.int32),
                                input_table, output_table)
    return _tc_loss(pos_sc, neg_sc)


# speedup vs baseline: 4.1021x; 1.0246x over previous
import functools

import jax
import jax.numpy as jnp
from jax import lax
from jax.experimental import pallas as pl
from jax.experimental.pallas import tpu as pltpu
from jax.experimental.pallas import tpu_sc as plsc

VOCAB = 1000000
DIM = 64
B = 16384
K = 20

NC = 2
NS = 16
NW = NC * NS
EW = B // NW
CB = 16
NCH = EW // CB
GPC = CB // 16
NEG_CH = CB * K
NBUF = 4
NEG_SPLITS = [(s, min(128, NEG_CH - s)) for s in range(0, NEG_CH, 128)]


def _sc_scores_kernel(center, context, neg, in_tab, out_tab,
                      pos_out, neg_out,
                      cen_idx, ctx_idx, neg_idx,
                      cen_rows, pos_rows, neg_rows,
                      pos_sc, neg_sc, *sems):
    wid = lax.axis_index("s") * NC + lax.axis_index("c")
    base = wid * EW

    pltpu.sync_copy(center.at[pl.ds(base, EW)], cen_idx)
    pltpu.sync_copy(context.at[pl.ds(base, EW)], ctx_idx)
    pltpu.sync_copy(neg.at[pl.ds(base * K, EW * K)], neg_idx)

    def fire(j, par):
        descs = [
            pltpu.make_async_copy(
                in_tab.at[cen_idx.at[pl.ds(j * CB, CB)]],
                cen_rows.at[par], sems[par]),
            pltpu.make_async_copy(
                out_tab.at[ctx_idx.at[pl.ds(j * CB, CB)]],
                pos_rows.at[par], sems[par]),
        ]
        for s, n in NEG_SPLITS:
            descs.append(pltpu.make_async_copy(
                out_tab.at[neg_idx.at[pl.ds(j * NEG_CH + s, n)]],
                neg_rows.at[par, pl.ds(s, n)], sems[par]))
        for d in descs:
            d.start()
        return descs

    iota = lax.iota(jnp.int32, 16)

    def compute(j, par):
        crows = cen_rows.at[par]
        prows = pos_rows.at[par]
        nrows = neg_rows.at[par]
        for g in range(GPC):
            row_ids = g * 16 + iota
            nbase = row_ids * K

            def body(d, carry):
                dcol = jnp.full((16,), d, dtype=jnp.int32)
                cd = plsc.load_gather(crows, [row_ids, dcol])
                pd = plsc.load_gather(prows, [row_ids, dcol])
                accs = [carry[0] + cd * pd]
                for k in range(K):
                    nd = plsc.load_gather(nrows, [nbase + k, dcol])
                    accs.append(carry[k + 1] + cd * nd)
                return tuple(accs)

            zeros = tuple(jnp.zeros((16,), jnp.float32) for _ in range(K + 1))
            accs = lax.fori_loop(0, DIM, body, zeros)

            off = j * CB + g * 16
            pos_sc[pl.ds(off, 16)] = accs[0]
            for k in range(K):
                neg_sc[k, pl.ds(off, 16)] = accs[k + 1]

    inflight = [fire(j, j % NBUF) for j in range(NBUF - 1)]
    for j in range(NCH):
        if j + NBUF - 1 < NCH:
            inflight.append(fire(j + NBUF - 1, (j + NBUF - 1) % NBUF))
        for d in inflight.pop(0):
            d.wait()
        compute(j, j % NBUF)

    pltpu.sync_copy(pos_sc, pos_out.at[pl.ds(base, EW)])
    pltpu.sync_copy(neg_sc, neg_out.at[wid])


@functools.partial(jax.jit, static_argnames=())
def _sc_scores(center, context, neg_flat, in_tab, out_tab):
    mesh = plsc.VectorSubcoreMesh(core_axis_name="c", subcore_axis_name="s")
    return pl.kernel(
        _sc_scores_kernel,
        out_type=(
            jax.ShapeDtypeStruct((B,), jnp.float32),
            jax.ShapeDtypeStruct((NW, K, EW), jnp.float32),
        ),
        mesh=mesh,
        scratch_types=(
            pltpu.VMEM((EW,), jnp.int32),
            pltpu.VMEM((EW,), jnp.int32),
            pltpu.VMEM((EW * K,), jnp.int32),
            pltpu.VMEM((NBUF, CB, DIM), jnp.float32),
            pltpu.VMEM((NBUF, CB, DIM), jnp.float32),
            pltpu.VMEM((NBUF, NEG_CH, DIM), jnp.float32),
            pltpu.VMEM((EW,), jnp.float32),
            pltpu.VMEM((K, EW), jnp.float32),
        ) + (pltpu.SemaphoreType.DMA,) * NBUF,
        compiler_params=pltpu.CompilerParams(needs_layout_passes=False,
                                             use_tc_tiling_on_sc=False),
        name="w2v_sc_scores",
    )(center, context, neg_flat, in_tab, out_tab)


def _tc_loss_kernel(pos_ref, neg_ref, out_ref):
    def ls(x):
        return jnp.minimum(x, 0.0) - jnp.log1p(jnp.exp(-jnp.abs(x)))

    total = jnp.sum(ls(pos_ref[...])) + jnp.sum(ls(-neg_ref[...]))
    out_ref[...] = jnp.full((1, 1), -1.0 / B) * total


def _tc_loss(pos, negs):
    pos2 = pos.reshape(128, 128)
    neg2 = negs.reshape(NW * K, EW)
    out = pl.pallas_call(
        _tc_loss_kernel,
        out_shape=jax.ShapeDtypeStruct((1, 1), jnp.float32),
    )(pos2, neg2)
    return out.reshape(())


def kernel(center, context, negatives, input_table, output_table):
    neg_flat = negatives.reshape(B * K)
    pos_sc, neg_sc = _sc_scores(center.astype(jnp.int32),
                                context.astype(jnp.int32),
                                neg_flat.astype(jnp.int32),
                                input_table, output_table)
    return _tc_loss(pos_sc, neg_sc)
